# baseline (speedup 1.0000x reference)
import functools

import jax
import jax.numpy as jnp
from jax import lax
from jax.experimental import pallas as pl
from jax.experimental.pallas import tpu as pltpu
from jax.experimental.pallas import tpu_sc as plsc

_P = 5
_BI = 128
_EI = 8


def _build_sc_kernel(B, L, ES, NW):
    TB = B // _BI
    TE = ES // _EI
    units = L * TB
    per_w = units // NW
    P = _P
    assert per_w % P == 0 and per_w >= 2 * P
    G = (per_w - P) // P
    mesh = plsc.VectorSubcoreMesh(core_axis_name="c", subcore_axis_name="s")
    scratch = (
        [pltpu.VMEM((_BI, ES), jnp.float32)] * P
        + [pltpu.VMEM((TE, _EI, _BI), jnp.float32)] * P
        + [pltpu.VMEM((_BI,), jnp.int32)] * (3 * P)
        + [pltpu.SemaphoreType.DMA] * (4 * P)
    )

    @functools.partial(
        pl.kernel,
        out_type=jax.ShapeDtypeStruct((L, TE, TB, _EI, _BI), jnp.float32),
        mesh=mesh,
        scratch_types=scratch,
        compiler_params=pltpu.CompilerParams(use_tc_tiling_on_sc=False,
                                             needs_layout_passes=False),
    )
    def k(nt4, gt4, li4, nte, gte, lne, out, *scr):
        rows = scr[0:P]
        tbufs = scr[P: 2 * P]
        idxs = [scr[2 * P + 3 * b: 2 * P + 3 * b + 3] for b in range(P)]
        si = scr[5 * P: 6 * P]
        sga = scr[6 * P: 7 * P]
        sbc = scr[7 * P: 8 * P]
        ss = scr[8 * P: 9 * P]
        wid = lax.axis_index("s") * 2 + lax.axis_index("c")
        u0 = wid * per_w
        iota16 = lax.iota(jnp.int32, 16)
        jvecs = [iota16 + (j * 16) for j in range(_BI // 16)]

        def unit_lb(u):
            l = u >> 5
            return l, u & (TB - 1)

        def scat_wait(b, u):
            l, tb = unit_lb(u)
            pltpu.make_async_copy(tbufs[b], out.at[l, :, tb], ss[b]).wait()

        def idx_slices(u):
            l, tb = unit_lb(u)
            tl = l >> 3
            li = l & 7
            return (nt4.at[tl, tb, li], gt4.at[tl, tb, li],
                    li4.at[tl, tb, li])

        def stage_a(b, u, wait_scat):
            if wait_scat:
                scat_wait(b, u - P)
            s0, s1, s2 = idx_slices(u)
            pltpu.async_copy(s0, idxs[b][0], si[b])
            pltpu.async_copy(s1, idxs[b][1], si[b])
            pltpu.async_copy(s2, idxs[b][2], si[b])

        def stage_g(b, u):
            s0, s1, s2 = idx_slices(u)
            pltpu.make_async_copy(s0, idxs[b][0], si[b]).wait()
            pltpu.make_async_copy(s1, idxs[b][1], si[b]).wait()
            pltpu.make_async_copy(s2, idxs[b][2], si[b]).wait()
            pltpu.async_copy(lne.at[idxs[b][2]], rows[b], sga[b])

        def stage_b(b):
            pltpu.make_async_copy(lne.at[idxs[b][2]], rows[b], sga[b]).wait()
            pltpu.async_copy(nte.at[idxs[b][0]], rows[b], sbc[b], add=True)
            pltpu.async_copy(gte.at[idxs[b][1]], rows[b], sbc[b], add=True)

        def stage_t(b, u):
            pltpu.make_async_copy(nte.at[idxs[b][0]], rows[b], sbc[b]).wait()
            pltpu.make_async_copy(gte.at[idxs[b][1]], rows[b], sbc[b]).wait()

            def te_body(te, carry):
                for ep in range(_EI // 2):
                    vs = []
                    for ei in (2 * ep, 2 * ep + 1):
                        e = te * _EI + ei
                        evec = jnp.zeros((16,), jnp.int32) + e
                        for j in range(_BI // 16):
                            vs.append(plsc.load_gather(rows[b],
                                                       [jvecs[j], evec]))
                    for k2, ei in enumerate((2 * ep, 2 * ep + 1)):
                        for j in range(_BI // 16):
                            tbufs[b][te, ei, pl.ds(j * 16, 16)] = (
                                vs[k2 * (_BI // 16) + j])
                return carry

            lax.fori_loop(0, TE, te_body, None)
            l, tb = unit_lb(u)
            pltpu.async_copy(tbufs[b], out.at[l, :, tb], ss[b])

        stage_a(0, u0 + 0, False)
        stage_a(1, u0 + 1, False)
        stage_g(0, u0 + 0)
        stage_a(2, u0 + 2, False)
        stage_g(1, u0 + 1)
        stage_b(0)
        stage_a(3, u0 + 3, False)
        stage_g(2, u0 + 2)
        stage_b(1)
        stage_t(0, u0 + 0)
        stage_a(4, u0 + 4, False)
        stage_g(3, u0 + 3)
        stage_b(2)
        stage_t(1, u0 + 1)

        def group(g, carry):
            ub = u0 + P + P * g
            for b in range(P):
                u = ub + b
                stage_a(b, u, True)
                stage_g((b + P - 1) % P, u - 1)
                stage_b((b + P - 2) % P)
                stage_t((b + P - 3) % P, u - 3)
            return carry

        lax.fori_loop(0, G, group, None)

        ul = u0 + per_w - 1
        stage_g((per_w - 1) % P, ul)
        stage_b((per_w - 2) % P)
        stage_t((per_w - 3) % P, ul - 2)
        stage_b((per_w - 1) % P)
        stage_t((per_w - 2) % P, ul - 1)
        stage_t((per_w - 1) % P, ul)
        for m in range(P):
            scat_wait((per_w - P + m) % P, ul - (P - 1) + m)

    return k


def kernel(naming_types, group_types, line_ids, naming_type_embeddings,
           group_type_embeddings, lines_num_embeddings):
    B, L = naming_types.shape
    ES = naming_type_embeddings.shape[1]
    NW = 32

    def idx_view(a):
        return (a.T.reshape(L // 8, 8, B // 128, 128)
                .transpose(0, 2, 1, 3).astype(jnp.int32))

    out5 = _build_sc_kernel(B, L, ES, NW)(
        idx_view(naming_types), idx_view(group_types), idx_view(line_ids),
        naming_type_embeddings, group_type_embeddings, lines_num_embeddings)
    return out5.transpose(2, 4, 0, 1, 3).reshape(B, L, ES)


# speedup vs baseline: 2.1801x; 2.1801x over previous
import functools

import jax
import jax.numpy as jnp
from jax import lax
from jax.experimental import pallas as pl
from jax.experimental.pallas import tpu as pltpu
from jax.experimental.pallas import tpu_sc as plsc

_P = 5
_BI = 128
_EI = 8


def _build_sc_kernel(B, L, ES, NW):
    TB = B // _BI
    TE = ES // _EI
    units = L * TB
    per_w = units // NW
    P = _P
    assert per_w % P == 0 and per_w >= 2 * P
    G = (per_w - P) // P
    mesh = plsc.VectorSubcoreMesh(core_axis_name="c", subcore_axis_name="s")
    scratch = (
        [pltpu.VMEM((_BI, ES), jnp.float32)] * P
        + [pltpu.VMEM((ES * _BI,), jnp.float32)] * P
        + [pltpu.VMEM((_BI,), jnp.int32)] * (3 * P)
        + [pltpu.SemaphoreType.DMA] * (4 * P)
    )

    @functools.partial(
        pl.kernel,
        out_type=jax.ShapeDtypeStruct((L, TE, TB, _EI * _BI), jnp.float32),
        mesh=mesh,
        scratch_types=scratch,
        compiler_params=pltpu.CompilerParams(use_tc_tiling_on_sc=False,
                                             needs_layout_passes=False),
    )
    def k(nt4, gt4, li4, nte, gte, lne, out, *scr):
        rows = scr[0:P]
        tbufs = scr[P: 2 * P]
        idxs = [scr[2 * P + 3 * b: 2 * P + 3 * b + 3] for b in range(P)]
        si = scr[5 * P: 6 * P]
        sga = scr[6 * P: 7 * P]
        sbc = scr[7 * P: 8 * P]
        ss = scr[8 * P: 9 * P]
        wid = lax.axis_index("s") * 2 + lax.axis_index("c")
        u0 = wid * per_w
        iota16 = lax.iota(jnp.int32, 16)
        jvecs = [iota16 + j * 16 for j in range(_BI // 16)]
        cd = [(d + iota16) & 15 for d in range(16)]
        sd = [((d + iota16) & 15) * _BI + iota16 for d in range(16)]

        def unit_lb(u):
            l = u >> 5
            return l, u & (TB - 1)

        def scat_wait(b, u):
            l, tb = unit_lb(u)
            for te in range(TE):
                pltpu.make_async_copy(
                    tbufs[b].at[pl.ds(te * _EI * _BI, _EI * _BI)],
                    out.at[l, te, tb], ss[b]).wait()

        def idx_slices(u):
            l, tb = unit_lb(u)
            tl = l >> 3
            li = l & 7
            return (nt4.at[tl, tb, li], gt4.at[tl, tb, li],
                    li4.at[tl, tb, li])

        def stage_a(b, u, wait_scat):
            if wait_scat:
                scat_wait(b, u - P)
            s0, s1, s2 = idx_slices(u)
            pltpu.async_copy(s0, idxs[b][0], si[b])
            pltpu.async_copy(s1, idxs[b][1], si[b])
            pltpu.async_copy(s2, idxs[b][2], si[b])

        def stage_g(b, u):
            s0, s1, s2 = idx_slices(u)
            pltpu.make_async_copy(s0, idxs[b][0], si[b]).wait()
            pltpu.make_async_copy(s1, idxs[b][1], si[b]).wait()
            pltpu.make_async_copy(s2, idxs[b][2], si[b]).wait()
            pltpu.async_copy(lne.at[idxs[b][2]], rows[b], sga[b])

        def stage_b(b):
            pltpu.make_async_copy(lne.at[idxs[b][2]], rows[b], sga[b]).wait()
            pltpu.async_copy(nte.at[idxs[b][0]], rows[b], sbc[b], add=True)
            pltpu.async_copy(gte.at[idxs[b][1]], rows[b], sbc[b], add=True)

        def stage_t(b, u):
            pltpu.make_async_copy(nte.at[idxs[b][0]], rows[b], sbc[b]).wait()
            pltpu.make_async_copy(gte.at[idxs[b][1]], rows[b], sbc[b]).wait()

            def e0_body(t4, carry):
                e0 = t4 * 16
                cvecs = [cd[d] + e0 for d in range(16)]
                for j in range(_BI // 16):
                    soff = e0 * _BI + j * 16
                    vs = [plsc.load_gather(rows[b], [jvecs[j], cvecs[d]])
                          for d in range(16)]
                    for d in range(16):
                        plsc.store_scatter(tbufs[b], [sd[d] + soff], vs[d])
                return carry

            lax.fori_loop(0, ES // 16, e0_body, None)
            l, tb = unit_lb(u)
            for te in range(TE):
                pltpu.async_copy(
                    tbufs[b].at[pl.ds(te * _EI * _BI, _EI * _BI)],
                    out.at[l, te, tb], ss[b])

        stage_a(0, u0 + 0, False)
        stage_a(1, u0 + 1, False)
        stage_g(0, u0 + 0)
        stage_a(2, u0 + 2, False)
        stage_g(1, u0 + 1)
        stage_b(0)
        stage_a(3, u0 + 3, False)
        stage_g(2, u0 + 2)
        stage_b(1)
        stage_t(0, u0 + 0)
        stage_a(4, u0 + 4, False)
        stage_g(3, u0 + 3)
        stage_b(2)
        stage_t(1, u0 + 1)

        def group(g, carry):
            ub = u0 + P + P * g
            for b in range(P):
                u = ub + b
                stage_a(b, u, True)
                stage_g((b + P - 1) % P, u - 1)
                stage_b((b + P - 2) % P)
                stage_t((b + P - 3) % P, u - 3)
            return carry

        lax.fori_loop(0, G, group, None)

        ul = u0 + per_w - 1
        stage_g((per_w - 1) % P, ul)
        stage_b((per_w - 2) % P)
        stage_t((per_w - 3) % P, ul - 2)
        stage_b((per_w - 1) % P)
        stage_t((per_w - 2) % P, ul - 1)
        stage_t((per_w - 1) % P, ul)
        for m in range(P):
            scat_wait((per_w - P + m) % P, ul - (P - 1) + m)

    return k


def kernel(naming_types, group_types, line_ids, naming_type_embeddings,
           group_type_embeddings, lines_num_embeddings):
    B, L = naming_types.shape
    ES = naming_type_embeddings.shape[1]
    NW = 32

    def idx_view(a):
        return (a.T.reshape(L // 8, 8, B // 128, 128)
                .transpose(0, 2, 1, 3).astype(jnp.int32))

    out4 = _build_sc_kernel(B, L, ES, NW)(
        idx_view(naming_types), idx_view(group_types), idx_view(line_ids),
        naming_type_embeddings, group_type_embeddings, lines_num_embeddings)
    out5 = out4.reshape(L, ES // _EI, B // _BI, _EI, _BI)
    return out5.transpose(2, 4, 0, 1, 3).reshape(B, L, ES)
